# baseline (speedup 1.0000x reference)
import jax
import jax.numpy as jnp
from jax import lax
from jax.experimental import pallas as pl
from jax.experimental.pallas import tpu as pltpu
from jax.experimental.pallas import tpu_sc as plsc

NUM_FIELDS = 26
FIELD_SIZE = 100000
EMBED_DIM = 64
BATCH = 16384
TOTAL_ROWS = NUM_FIELDS * FIELD_SIZE

_info = plsc.get_sparse_core_info()
NC, NS, L = _info.num_cores, _info.num_subcores, _info.num_lanes
NW = NC * NS

N_UNITS = NUM_FIELDS * EMBED_DIM
U_PER_W = N_UNITS // NW
CHUNK = 2048
N_CHUNK = BATCH // CHUNK
CROWS = CHUNK // 128


def _gather_body(cat_hbm, w_hbm, out_hbm, slab, idxA, idxB, obA, obB,
                 ssem, isem0, isem1, osem0, osem1):
    wid = lax.axis_index("s") * NC + lax.axis_index("c")
    idxb = (idxA, idxB)
    obb = (obA, obB)
    isem = (isem0, isem1)
    osem = (osem0, osem1)

    def out_dst(f, ct, cl, k):
        return out_hbm.at[f, ct, pl.ds(k * CROWS, CROWS),
                          pl.ds(cl * 128, 128)]

    def unit_body(u, carry):
        f, c = carry
        ct = lax.shift_right_logical(c, 3)
        cl = lax.bitwise_and(c, 7)
        src = w_hbm.at[c, pl.ds(f * FIELD_SIZE, FIELD_SIZE)]
        pltpu.async_copy(src, slab, ssem)
        ibase = f * BATCH
        pltpu.async_copy(cat_hbm.at[pl.ds(ibase, CHUNK)], idxA, isem0)
        pltpu.make_async_copy(src, slab, ssem).wait()

        for k2 in range(N_CHUNK // 2):
            for h in range(2):
                k = k2 * 2 + h
                ib, ob = idxb[h], obb[h]
                pltpu.make_async_copy(
                    cat_hbm.at[pl.ds(ibase + k * CHUNK, CHUNK)],
                    ib, isem[h]).wait()
                if k + 1 < N_CHUNK:
                    pltpu.async_copy(
                        cat_hbm.at[pl.ds(ibase + (k + 1) * CHUNK, CHUNK)],
                        idxb[1 - h], isem[1 - h])

                for r in range(CROWS):
                    def gather(m, _, r=r, ib=ib, ob=ob):
                        o = pl.multiple_of(m * L, L)
                        ob[r, pl.ds(o, L)] = plsc.load_gather(
                            slab, [ib[pl.ds(r * 128 + o, L)]])
                        return 0
                    lax.fori_loop(0, 128 // L, gather, 0)

                if k >= 2:
                    pltpu.make_async_copy(ob, out_dst(f, ct, cl, k - 2),
                                          osem[h]).wait()
                pltpu.async_copy(ob, out_dst(f, ct, cl, k), osem[h])
        for h in range(2):
            k = N_CHUNK - 2 + h
            pltpu.make_async_copy(obb[h], out_dst(f, ct, cl, k),
                                  osem[h]).wait()
        c2 = lax.bitwise_and(c + 1, EMBED_DIM - 1)
        f2 = f + lax.shift_right_logical(c + 1, 6)
        return f2, c2

    u0 = wid * U_PER_W
    f0 = lax.shift_right_logical(u0, 6)
    c0 = lax.bitwise_and(u0, EMBED_DIM - 1)
    lax.fori_loop(0, U_PER_W, unit_body, (f0, c0))


@jax.jit
def kernel(categorical_inputs, weights):
    w_t = weights.T
    cat_f = categorical_inputs.T.reshape(-1)
    mesh = plsc.VectorSubcoreMesh(core_axis_name="c", subcore_axis_name="s")
    out4 = pl.kernel(
        _gather_body,
        out_type=jax.ShapeDtypeStruct(
            (NUM_FIELDS, EMBED_DIM // 8, BATCH // 128, 8 * 128), jnp.float32),
        mesh=mesh,
        scratch_types=[
            pltpu.VMEM((FIELD_SIZE,), jnp.float32),
            pltpu.VMEM((CHUNK,), jnp.int32),
            pltpu.VMEM((CHUNK,), jnp.int32),
            pltpu.VMEM((CROWS, 128), jnp.float32),
            pltpu.VMEM((CROWS, 128), jnp.float32),
            pltpu.SemaphoreType.DMA,
            pltpu.SemaphoreType.DMA,
            pltpu.SemaphoreType.DMA,
            pltpu.SemaphoreType.DMA,
            pltpu.SemaphoreType.DMA,
        ],
        compiler_params=pltpu.CompilerParams(use_tc_tiling_on_sc=False,
                                             needs_layout_passes=False),
    )(cat_f, w_t)
    out5 = out4.reshape(NUM_FIELDS, EMBED_DIM // 8, BATCH // 128, 8, 128)
    return out5.transpose(2, 4, 0, 1, 3).reshape(BATCH, NUM_FIELDS, EMBED_DIM)


# speedup vs baseline: 7.0526x; 7.0526x over previous
import jax
import jax.numpy as jnp
from jax import lax
from jax.experimental import pallas as pl
from jax.experimental.pallas import tpu as pltpu
from jax.experimental.pallas import tpu_sc as plsc

NUM_FIELDS = 26
FIELD_SIZE = 100000
EMBED_DIM = 64
BATCH = 16384
TOTAL_ROWS = NUM_FIELDS * FIELD_SIZE

_info = plsc.get_sparse_core_info()
NC, NS, L = _info.num_cores, _info.num_subcores, _info.num_lanes
NW = NC * NS

ROWS_PER_W = BATCH // NW
BLK_ROWS = 32
N_BLK = ROWS_PER_W // BLK_ROWS
BLK_LOOK = BLK_ROWS * NUM_FIELDS
JROWS = 13
JCOLS = BLK_LOOK // JROWS


def _tec_body(cat_hbm, w_hbm, out_hbm, ib2, jidx, rows,
              isem0, isem1, gsem0, gsem1, osem0, osem1):
    wid = lax.axis_index("s") * NC + lax.axis_index("c")
    base_row = wid * ROWS_PER_W
    base_look = base_row * NUM_FIELDS
    iota = lax.iota(jnp.int32, L)
    isem = (isem0, isem1)
    gsem = (gsem0, gsem1)
    osem = (osem0, osem1)

    def idx_start(i, h):
        pltpu.async_copy(cat_hbm.at[pl.ds(base_row + i * BLK_ROWS, BLK_ROWS)],
                         ib2.at[h], isem[h])

    def idx_wait(i, h):
        pltpu.make_async_copy(
            cat_hbm.at[pl.ds(base_row + i * BLK_ROWS, BLK_ROWS)],
            ib2.at[h], isem[h]).wait()

    def compute_jidx(h):
        for r in range(JROWS):
            def lane(m, _, r=r):
                o = pl.multiple_of(m * L, L)
                p16 = r * JCOLS + o + iota
                prow = lax.div(p16, NUM_FIELDS)
                pcol = p16 - prow * NUM_FIELDS
                raw = plsc.load_gather(ib2.at[h], [prow, pcol])
                jidx[h, r, pl.ds(o, L)] = raw + pcol * FIELD_SIZE
                return 0
            lax.fori_loop(0, JCOLS // L, lane, 0)

    def gather_start(h):
        for r in range(JROWS):
            pltpu.async_copy(w_hbm.at[jidx.at[h, r]],
                             rows.at[h, pl.ds(r * JCOLS, JCOLS)], gsem[h])

    def gather_wait(h):
        for r in range(JROWS):
            pltpu.make_async_copy(w_hbm.at[jidx.at[h, r]],
                                  rows.at[h, pl.ds(r * JCOLS, JCOLS)],
                                  gsem[h]).wait()

    def out_start(i, h):
        pltpu.async_copy(rows.at[h],
                         out_hbm.at[pl.ds(base_look + i * BLK_LOOK, BLK_LOOK)],
                         osem[h])

    def out_wait(i, h):
        pltpu.make_async_copy(
            rows.at[h],
            out_hbm.at[pl.ds(base_look + i * BLK_LOOK, BLK_LOOK)],
            osem[h]).wait()

    idx_start(0, 0)
    idx_wait(0, 0)
    compute_jidx(0)
    gather_start(0)
    idx_start(1, 1)

    for i in range(N_BLK):
        h = i % 2
        h2 = 1 - h
        if i + 1 < N_BLK:
            idx_wait(i + 1, h2)
            compute_jidx(h2)
        gather_wait(h)
        if i >= 1:
            out_wait(i - 1, h2)
        if i + 1 < N_BLK:
            gather_start(h2)
        out_start(i, h)
        if i + 2 < N_BLK:
            idx_start(i + 2, h)
    out_wait(N_BLK - 1, (N_BLK - 1) % 2)


@jax.jit
def kernel(categorical_inputs, weights):
    mesh = plsc.VectorSubcoreMesh(core_axis_name="c", subcore_axis_name="s")
    out2 = pl.kernel(
        _tec_body,
        out_type=jax.ShapeDtypeStruct((BATCH * NUM_FIELDS, EMBED_DIM),
                                      jnp.float32),
        mesh=mesh,
        scratch_types=[
            pltpu.VMEM((2, BLK_ROWS, NUM_FIELDS), jnp.int32),
            pltpu.VMEM((2, JROWS, JCOLS), jnp.int32),
            pltpu.VMEM((2, BLK_LOOK, EMBED_DIM), jnp.float32),
            pltpu.SemaphoreType.DMA,
            pltpu.SemaphoreType.DMA,
            pltpu.SemaphoreType.DMA,
            pltpu.SemaphoreType.DMA,
            pltpu.SemaphoreType.DMA,
            pltpu.SemaphoreType.DMA,
        ],
        compiler_params=pltpu.CompilerParams(use_tc_tiling_on_sc=False,
                                             needs_layout_passes=False),
    )(categorical_inputs, weights)
    return out2.reshape(BATCH, NUM_FIELDS, EMBED_DIM)


# speedup vs baseline: 7.0668x; 1.0020x over previous
import jax
import jax.numpy as jnp
from jax import lax
from jax.experimental import pallas as pl
from jax.experimental.pallas import tpu as pltpu
from jax.experimental.pallas import tpu_sc as plsc

NUM_FIELDS = 26
FIELD_SIZE = 100000
EMBED_DIM = 64
BATCH = 16384
TOTAL_ROWS = NUM_FIELDS * FIELD_SIZE

_info = plsc.get_sparse_core_info()
NC, NS, L = _info.num_cores, _info.num_subcores, _info.num_lanes
NW = NC * NS

ROWS_PER_W = BATCH // NW
BLK_ROWS = 32
N_BLK = ROWS_PER_W // BLK_ROWS
BLK_LOOK = BLK_ROWS * NUM_FIELDS
JROWS = 13
JCOLS = BLK_LOOK // JROWS


def _tec_body(cat_hbm, w_hbm, out_hbm, ib2, jidx, rows,
              isem0, isem1, gsem0, gsem1, osem0, osem1):
    wid = lax.axis_index("s") * NC + lax.axis_index("c")
    base_row = wid * ROWS_PER_W
    base_look = base_row * NUM_FIELDS
    iota = lax.iota(jnp.int32, L)
    isem = (isem0, isem1)
    gsem = (gsem0, gsem1)
    osem = (osem0, osem1)

    def idx_start(i, h):
        pltpu.async_copy(
            cat_hbm.at[pl.ds(base_look + i * BLK_LOOK, BLK_LOOK)],
            ib2.at[h], isem[h])

    def idx_wait(i, h):
        pltpu.make_async_copy(
            cat_hbm.at[pl.ds(base_look + i * BLK_LOOK, BLK_LOOK)],
            ib2.at[h], isem[h]).wait()

    def compute_jidx(h):
        for r in range(JROWS):
            def lane(m, _, r=r):
                o = pl.multiple_of(m * L, L)
                pcol = lax.rem(r * JCOLS + o + iota, NUM_FIELDS)
                raw = ib2[h, pl.ds(r * JCOLS + o, L)]
                jidx[h, r, pl.ds(o, L)] = raw + pcol * FIELD_SIZE
                return 0
            lax.fori_loop(0, JCOLS // L, lane, 0)

    def gather_start(h):
        for r in range(JROWS):
            pltpu.async_copy(w_hbm.at[jidx.at[h, r]],
                             rows.at[h, pl.ds(r * JCOLS, JCOLS)], gsem[h])

    def gather_wait(h):
        for r in range(JROWS):
            pltpu.make_async_copy(w_hbm.at[jidx.at[h, r]],
                                  rows.at[h, pl.ds(r * JCOLS, JCOLS)],
                                  gsem[h]).wait()

    def out_start(i, h):
        pltpu.async_copy(rows.at[h],
                         out_hbm.at[pl.ds(base_look + i * BLK_LOOK, BLK_LOOK)],
                         osem[h])

    def out_wait(i, h):
        pltpu.make_async_copy(
            rows.at[h],
            out_hbm.at[pl.ds(base_look + i * BLK_LOOK, BLK_LOOK)],
            osem[h]).wait()

    idx_start(0, 0)
    idx_wait(0, 0)
    compute_jidx(0)
    gather_start(0)
    idx_start(1, 1)

    for i in range(N_BLK):
        h = i % 2
        h2 = 1 - h
        if i + 1 < N_BLK:
            idx_wait(i + 1, h2)
            compute_jidx(h2)
        gather_wait(h)
        if i >= 1:
            out_wait(i - 1, h2)
        if i + 1 < N_BLK:
            gather_start(h2)
        out_start(i, h)
        if i + 2 < N_BLK:
            idx_start(i + 2, h)
    out_wait(N_BLK - 1, (N_BLK - 1) % 2)


@jax.jit
def kernel(categorical_inputs, weights):
    mesh = plsc.VectorSubcoreMesh(core_axis_name="c", subcore_axis_name="s")
    out2 = pl.kernel(
        _tec_body,
        out_type=jax.ShapeDtypeStruct((BATCH * NUM_FIELDS, EMBED_DIM),
                                      jnp.float32),
        mesh=mesh,
        scratch_types=[
            pltpu.VMEM((2, BLK_LOOK), jnp.int32),
            pltpu.VMEM((2, JROWS, JCOLS), jnp.int32),
            pltpu.VMEM((2, BLK_LOOK, EMBED_DIM), jnp.float32),
            pltpu.SemaphoreType.DMA,
            pltpu.SemaphoreType.DMA,
            pltpu.SemaphoreType.DMA,
            pltpu.SemaphoreType.DMA,
            pltpu.SemaphoreType.DMA,
            pltpu.SemaphoreType.DMA,
        ],
        compiler_params=pltpu.CompilerParams(use_tc_tiling_on_sc=False,
                                             needs_layout_passes=False),
    )(categorical_inputs.reshape(-1), weights)
    return out2.reshape(BATCH, NUM_FIELDS, EMBED_DIM)


# speedup vs baseline: 7.0746x; 1.0011x over previous
import jax
import jax.numpy as jnp
from jax import lax
from jax.experimental import pallas as pl
from jax.experimental.pallas import tpu as pltpu
from jax.experimental.pallas import tpu_sc as plsc

NUM_FIELDS = 26
FIELD_SIZE = 100000
EMBED_DIM = 64
BATCH = 16384
TOTAL_ROWS = NUM_FIELDS * FIELD_SIZE

_info = plsc.get_sparse_core_info()
NC, NS, L = _info.num_cores, _info.num_subcores, _info.num_lanes
NW = NC * NS

ROWS_PER_W = BATCH // NW
BLK_ROWS = 32
N_BLK = ROWS_PER_W // BLK_ROWS
BLK_LOOK = BLK_ROWS * NUM_FIELDS
JROWS = 13
JCOLS = BLK_LOOK // JROWS


def _tec_body(cat_hbm, w_hbm, out_hbm, jidx, rows,
              isem0, isem1, gsem0, gsem1, osem0, osem1):
    wid = lax.axis_index("s") * NC + lax.axis_index("c")
    base_look = wid * ROWS_PER_W * NUM_FIELDS
    isem = (isem0, isem1)
    gsem = (gsem0, gsem1)
    osem = (osem0, osem1)

    def idx_start(i, h):
        pltpu.async_copy(cat_hbm.at[wid * N_BLK + i], jidx.at[h], isem[h])

    def idx_wait(i, h):
        pltpu.make_async_copy(cat_hbm.at[wid * N_BLK + i], jidx.at[h],
                              isem[h]).wait()

    def gather_start(h):
        for r in range(JROWS):
            pltpu.async_copy(w_hbm.at[jidx.at[h, r]],
                             rows.at[h, pl.ds(r * JCOLS, JCOLS)], gsem[h])

    def gather_wait(h):
        for r in range(JROWS):
            pltpu.make_async_copy(w_hbm.at[jidx.at[h, r]],
                                  rows.at[h, pl.ds(r * JCOLS, JCOLS)],
                                  gsem[h]).wait()

    def out_start(i, h):
        pltpu.async_copy(rows.at[h],
                         out_hbm.at[pl.ds(base_look + i * BLK_LOOK, BLK_LOOK)],
                         osem[h])

    def out_wait(i, h):
        pltpu.make_async_copy(
            rows.at[h],
            out_hbm.at[pl.ds(base_look + i * BLK_LOOK, BLK_LOOK)],
            osem[h]).wait()

    idx_start(0, 0)
    idx_wait(0, 0)
    gather_start(0)
    idx_start(1, 1)

    for i in range(N_BLK):
        h = i % 2
        h2 = 1 - h
        if i + 1 < N_BLK:
            idx_wait(i + 1, h2)
        gather_wait(h)
        if i >= 1:
            out_wait(i - 1, h2)
        if i + 1 < N_BLK:
            gather_start(h2)
        out_start(i, h)
        if i + 2 < N_BLK:
            idx_start(i + 2, h)
    out_wait(N_BLK - 1, (N_BLK - 1) % 2)


@jax.jit
def kernel(categorical_inputs, weights):
    offs = jnp.arange(NUM_FIELDS, dtype=categorical_inputs.dtype) * FIELD_SIZE
    joint3 = (categorical_inputs + offs[None, :]).reshape(
        NW * N_BLK, JROWS, JCOLS)
    mesh = plsc.VectorSubcoreMesh(core_axis_name="c", subcore_axis_name="s")
    out2 = pl.kernel(
        _tec_body,
        out_type=jax.ShapeDtypeStruct((BATCH * NUM_FIELDS, EMBED_DIM),
                                      jnp.float32),
        mesh=mesh,
        scratch_types=[
            pltpu.VMEM((2, JROWS, JCOLS), jnp.int32),
            pltpu.VMEM((2, BLK_LOOK, EMBED_DIM), jnp.float32),
            pltpu.SemaphoreType.DMA,
            pltpu.SemaphoreType.DMA,
            pltpu.SemaphoreType.DMA,
            pltpu.SemaphoreType.DMA,
            pltpu.SemaphoreType.DMA,
            pltpu.SemaphoreType.DMA,
        ],
        compiler_params=pltpu.CompilerParams(use_tc_tiling_on_sc=False,
                                             needs_layout_passes=False),
    )(joint3, weights)
    return out2.reshape(BATCH, NUM_FIELDS, EMBED_DIM)
